# Initial kernel scaffold; baseline (speedup 1.0000x reference)
import functools

import jax
import jax.numpy as jnp
from jax import lax
from jax.experimental import pallas as pl
from jax.experimental.pallas import tpu as pltpu
from jax.experimental.pallas import tpu_sc as plsc

NN = 10000
NE = 320000
F = 128
NG = 64
NCLS = 40
NW = 32
CHUNK = 128
CPT = 79
EPAD = NW * CPT * CHUNK
NROW = 10016
RPT = NROW // 16
RPT2 = RPT // 2

_mesh = plsc.VectorSubcoreMesh(core_axis_name="c", subcore_axis_name="s")


@functools.partial(
    pl.kernel,
    out_type=jax.ShapeDtypeStruct((2, NROW, 16), jnp.float32),
    mesh=_mesh,
    scratch_types=[
        pltpu.VMEM((CPT, CHUNK), jnp.int32),
        pltpu.VMEM((CHUNK, 16), jnp.float32),
        pltpu.VMEM((RPT, 16), jnp.float32),
        pltpu.VMEM_SHARED((NROW, 16), jnp.float32),
    ],
)
def _deg_kernel(dst_hbm, out_hbm, didx, ones_v, zbuf, acc_sh):
    c = lax.axis_index("c")
    s = lax.axis_index("s")
    wid = s * 2 + c
    one_row = jnp.where(lax.iota(jnp.int32, 16) == 0, 1.0, 0.0)
    zero_row = jnp.zeros((16,), jnp.float32)

    def init_ones(i, carry):
        ones_v[i, :] = one_row
        return carry

    lax.fori_loop(0, CHUNK, init_ones, 0)

    def init_zero(i, carry):
        zbuf[i, :] = zero_row
        return carry

    lax.fori_loop(0, RPT, init_zero, 0)
    pltpu.sync_copy(zbuf, acc_sh.at[pl.ds(s * RPT, RPT)])
    plsc.subcore_barrier()

    pltpu.sync_copy(dst_hbm.at[wid], didx)

    def body(t, carry):
        pltpu.sync_copy(ones_v, acc_sh.at[didx.at[t]], add=True)
        return carry

    lax.fori_loop(0, CPT, body, 0)
    plsc.subcore_barrier()
    pltpu.sync_copy(acc_sh.at[pl.ds(s * RPT, RPT)], zbuf)
    pltpu.sync_copy(zbuf, out_hbm.at[c, pl.ds(s * RPT, RPT)])


@functools.partial(
    pl.kernel,
    out_type=jax.ShapeDtypeStruct((2, NROW, F), jnp.float32),
    mesh=_mesh,
    scratch_types=[
        pltpu.VMEM((CPT, CHUNK), jnp.int32),
        pltpu.VMEM((CPT, CHUNK), jnp.int32),
        pltpu.VMEM((CHUNK, F), jnp.float32),
        pltpu.VMEM((RPT2, F), jnp.float32),
        pltpu.VMEM_SHARED((NROW, F), jnp.float32),
        pltpu.SemaphoreType.DMA,
    ],
)
def _prop_kernel(g_hbm, src_hbm, dst_hbm, out_hbm, sidx, didx, rows, zbuf,
                 acc_sh, sem):
    c = lax.axis_index("c")
    s = lax.axis_index("s")
    wid = s * 2 + c
    zero_row = jnp.zeros((16,), jnp.float32)

    def init_zero(i, carry):
        for j in range(F // 16):
            zbuf[i, pl.ds(j * 16, 16)] = zero_row
        return carry

    lax.fori_loop(0, RPT2, init_zero, 0)
    pltpu.sync_copy(zbuf, acc_sh.at[pl.ds(s * RPT, RPT2)])
    pltpu.sync_copy(zbuf, acc_sh.at[pl.ds(s * RPT + RPT2, RPT2)])
    plsc.subcore_barrier()

    pltpu.sync_copy(src_hbm.at[wid], sidx)
    pltpu.sync_copy(dst_hbm.at[wid], didx)

    def body(t, carry):
        pltpu.async_copy(g_hbm.at[sidx.at[t]], rows, sem).wait()
        pltpu.sync_copy(rows, acc_sh.at[didx.at[t]], add=True)
        return carry

    lax.fori_loop(0, CPT, body, 0)
    plsc.subcore_barrier()
    for half in range(2):
        pltpu.sync_copy(acc_sh.at[pl.ds(s * RPT + half * RPT2, RPT2)], zbuf)
        pltpu.sync_copy(zbuf, out_hbm.at[c, pl.ds(s * RPT + half * RPT2, RPT2)])


def _tc1_body(x_ref, w_ref, degp_ref, g_ref, dinv_ref):
    d = degp_ref[0, :, 0:1] + degp_ref[1, :, 0:1] + 1.0
    dinv = lax.rsqrt(d[:NN])
    dinv_ref[...] = dinv
    h = jnp.dot(x_ref[...], w_ref[...], preferred_element_type=jnp.float32)
    g_ref[...] = dinv * h


_tc1 = pl.pallas_call(
    _tc1_body,
    out_shape=[
        jax.ShapeDtypeStruct((NN, F), jnp.float32),
        jax.ShapeDtypeStruct((NN, 1), jnp.float32),
    ],
)


def _tc_mid_body(p_ref, g_ref, dinv_ref, b_ref, w_ref, out_ref):
    agg = p_ref[0, :NN, :] + p_ref[1, :NN, :] + g_ref[...]
    xn = jnp.maximum(dinv_ref[...] * agg + b_ref[...], 0.0)
    h = jnp.dot(xn, w_ref[...], preferred_element_type=jnp.float32)
    out_ref[...] = dinv_ref[...] * h


_tc_mid = pl.pallas_call(
    _tc_mid_body,
    out_shape=jax.ShapeDtypeStruct((NN, F), jnp.float32),
)


def _tc_fin_body(p_ref, g_ref, dinv_ref, b_ref, batch_ref, wl_ref, bl_ref,
                 out_ref):
    agg = p_ref[0, :NN, :] + p_ref[1, :NN, :] + g_ref[...]
    x4 = dinv_ref[...] * agg + b_ref[...]
    gid = lax.broadcasted_iota(jnp.int32, (NG, NN), 0)
    m = (batch_ref[...] == gid).astype(jnp.float32)
    sums = jnp.dot(m, x4, preferred_element_type=jnp.float32)
    counts = jnp.sum(m, axis=1, keepdims=True)
    pooled = sums / jnp.maximum(counts, 1.0)
    out_ref[...] = (
        jnp.dot(pooled, wl_ref[...], preferred_element_type=jnp.float32)
        + bl_ref[...]
    )


_tc_fin = pl.pallas_call(
    _tc_fin_body,
    out_shape=jax.ShapeDtypeStruct((NG, F), jnp.float32),
)


def kernel(x, edge_index, batch, W1, b1, W2, b2, W3, b3, Wl, bl):
    src = edge_index[0].astype(jnp.int32)
    dst = edge_index[1].astype(jnp.int32)
    pad = EPAD - NE
    srcp = jnp.concatenate([src, jnp.zeros((pad,), jnp.int32)])
    srcp = srcp.reshape(NW, CPT, CHUNK)
    dstp = jnp.concatenate([dst, jnp.full((pad,), NN, jnp.int32)])
    dstp = dstp.reshape(NW, CPT, CHUNK)

    degp = _deg_kernel(dstp)
    g1, dinv = _tc1(x, W1, degp)
    p1 = _prop_kernel(g1, srcp, dstp)
    g2 = _tc_mid(p1, g1, dinv, b1.reshape(1, F), W2)
    p2 = _prop_kernel(g2, srcp, dstp)
    g3 = _tc_mid(p2, g2, dinv, b2.reshape(1, F), W3)
    p3 = _prop_kernel(g3, srcp, dstp)

    wlp = jnp.pad(Wl, ((0, 0), (0, F - NCLS)))
    blp = jnp.pad(bl, (0, F - NCLS)).reshape(1, F)
    out = _tc_fin(p3, g3, dinv, b3.reshape(1, F),
                  batch.astype(jnp.int32).reshape(1, NN), wlp, blp)
    return out[:, :NCLS]


# speedup vs baseline: 3.5855x; 3.5855x over previous
import functools

import jax
import jax.numpy as jnp
from jax import lax
from jax.experimental import pallas as pl
from jax.experimental.pallas import tpu as pltpu
from jax.experimental.pallas import tpu_sc as plsc

NN = 10000
NE = 320000
F = 128
NG = 64
NCLS = 40
NS = 16
CHUNK = 128
CPS = 160
EPAD = NS * CPS * CHUNK
HALF = 5120
NROWH = 5248
RPTH = NROWH // NS

_mesh = plsc.VectorSubcoreMesh(core_axis_name="c", subcore_axis_name="s")


@functools.partial(
    pl.kernel,
    out_type=jax.ShapeDtypeStruct((2, NROWH, F), jnp.float32),
    mesh=_mesh,
    scratch_types=[
        pltpu.VMEM((CPS, CHUNK), jnp.int32),
        pltpu.VMEM((CPS, CHUNK), jnp.int32),
        pltpu.VMEM((CHUNK, F), jnp.float32),
        pltpu.VMEM_SHARED((NROWH, F), jnp.float32),
        pltpu.SemaphoreType.DMA,
    ],
)
def _prop_kernel(g_hbm, src_hbm, dst_hbm, out_hbm, sidx, didx, rows,
                 acc_sh, sem):
    c = lax.axis_index("c")
    s = lax.axis_index("s")
    zero_row = jnp.zeros((16,), jnp.float32)

    def init_zero(i, carry):
        for j in range(F // 16):
            rows[i, pl.ds(j * 16, 16)] = zero_row
        return carry

    lax.fori_loop(0, CHUNK, init_zero, 0)
    for off, n in ((0, CHUNK), (CHUNK, CHUNK), (2 * CHUNK, RPTH - 2 * CHUNK)):
        pltpu.sync_copy(rows.at[pl.ds(0, n)],
                        acc_sh.at[pl.ds(s * RPTH + off, n)])
    plsc.subcore_barrier()

    pltpu.sync_copy(src_hbm.at[s], sidx)
    pltpu.sync_copy(dst_hbm.at[s], didx)

    base = c * HALF

    def remap(t, carry):
        for j in range(CHUNK // 16):
            v = didx[t, pl.ds(j * 16, 16)] - base
            ok = (v >= 0) & (v < HALF)
            didx[t, pl.ds(j * 16, 16)] = jnp.where(ok, v, HALF)
        return carry

    lax.fori_loop(0, CPS, remap, 0)

    def body(t, carry):
        pltpu.async_copy(g_hbm.at[sidx.at[t]], rows, sem).wait()
        pltpu.sync_copy(rows, acc_sh.at[didx.at[t]], add=True)
        return carry

    lax.fori_loop(0, CPS, body, 0)
    plsc.subcore_barrier()
    for off, n in ((0, CHUNK), (CHUNK, CHUNK), (2 * CHUNK, RPTH - 2 * CHUNK)):
        pltpu.sync_copy(acc_sh.at[pl.ds(s * RPTH + off, n)],
                        rows.at[pl.ds(0, n)])
        pltpu.sync_copy(rows.at[pl.ds(0, n)],
                        out_hbm.at[c, pl.ds(s * RPTH + off, n)])


def _tc1_body(x_ref, w_ref, degp_ref, g_ref, dinv_ref):
    d = jnp.concatenate(
        [degp_ref[0, :HALF, 0:1], degp_ref[1, :NN - HALF, 0:1]],
        axis=0) + 1.0
    dinv = lax.rsqrt(d)
    dinv_ref[...] = dinv
    h = jnp.dot(x_ref[...], w_ref[...], preferred_element_type=jnp.float32)
    g_ref[...] = dinv * h


_tc1 = pl.pallas_call(
    _tc1_body,
    out_shape=[
        jax.ShapeDtypeStruct((NN, F), jnp.float32),
        jax.ShapeDtypeStruct((NN, 1), jnp.float32),
    ],
)


def _agg(p_ref, g_ref):
    return jnp.concatenate(
        [p_ref[0, :HALF, :], p_ref[1, :NN - HALF, :]], axis=0) + g_ref[...]


def _tc_mid_body(p_ref, g_ref, dinv_ref, b_ref, w_ref, out_ref):
    xn = jnp.maximum(dinv_ref[...] * _agg(p_ref, g_ref) + b_ref[...], 0.0)
    h = jnp.dot(xn, w_ref[...], preferred_element_type=jnp.float32)
    out_ref[...] = dinv_ref[...] * h


_tc_mid = pl.pallas_call(
    _tc_mid_body,
    out_shape=jax.ShapeDtypeStruct((NN, F), jnp.float32),
)


def _tc_fin_body(p_ref, g_ref, dinv_ref, b_ref, batch_ref, wl_ref, bl_ref,
                 out_ref):
    x4 = dinv_ref[...] * _agg(p_ref, g_ref) + b_ref[...]
    gid = lax.broadcasted_iota(jnp.int32, (NG, NN), 0)
    m = (batch_ref[...] == gid).astype(jnp.float32)
    sums = jnp.dot(m, x4, preferred_element_type=jnp.float32)
    counts = jnp.sum(m, axis=1, keepdims=True)
    pooled = sums / jnp.maximum(counts, 1.0)
    out_ref[...] = (
        jnp.dot(pooled, wl_ref[...], preferred_element_type=jnp.float32)
        + bl_ref[...]
    )


_tc_fin = pl.pallas_call(
    _tc_fin_body,
    out_shape=jax.ShapeDtypeStruct((NG, F), jnp.float32),
)


def kernel(x, edge_index, batch, W1, b1, W2, b2, W3, b3, Wl, bl):
    src = edge_index[0].astype(jnp.int32)
    dst = edge_index[1].astype(jnp.int32)
    pad = EPAD - NE
    srcp = jnp.concatenate([src, jnp.zeros((pad,), jnp.int32)])
    srcp = srcp.reshape(NS, CPS, CHUNK)
    dstp = jnp.concatenate([dst, jnp.full((pad,), NN, jnp.int32)])
    dstp = dstp.reshape(NS, CPS, CHUNK)

    degp = _prop_kernel(jnp.ones((NN, F), jnp.float32), srcp, dstp)
    g1, dinv = _tc1(x, W1, degp)
    p1 = _prop_kernel(g1, srcp, dstp)
    g2 = _tc_mid(p1, g1, dinv, b1.reshape(1, F), W2)
    p2 = _prop_kernel(g2, srcp, dstp)
    g3 = _tc_mid(p2, g2, dinv, b2.reshape(1, F), W3)
    p3 = _prop_kernel(g3, srcp, dstp)

    wlp = jnp.pad(Wl, ((0, 0), (0, F - NCLS)))
    blp = jnp.pad(bl, (0, F - NCLS)).reshape(1, F)
    out = _tc_fin(p3, g3, dinv, b3.reshape(1, F),
                  batch.astype(jnp.int32).reshape(1, NN), wlp, blp)
    return out[:, :NCLS]


# speedup vs baseline: 4.6024x; 1.2836x over previous
import functools

import jax
import jax.numpy as jnp
from jax import lax
from jax.experimental import pallas as pl
from jax.experimental.pallas import tpu as pltpu
from jax.experimental.pallas import tpu_sc as plsc

NN = 10000
NE = 320000
F = 128
NG = 64
NCLS = 40
NS = 16
CHUNK = 128
CPS = 160
EPAD = NS * CPS * CHUNK
HALF = 5120
NROWH = 5248
RPTH = NROWH // NS

_mesh = plsc.VectorSubcoreMesh(core_axis_name="c", subcore_axis_name="s")


def _zero_acc(rows, acc_sh, s):
    zero_row = jnp.zeros((16,), jnp.float32)

    def init_zero(i, carry):
        for j in range(F // 16):
            rows[i, pl.ds(j * 16, 16)] = zero_row
        return carry

    lax.fori_loop(0, CHUNK, init_zero, 0)
    for off, n in ((0, CHUNK), (CHUNK, CHUNK), (2 * CHUNK, RPTH - 2 * CHUNK)):
        pltpu.sync_copy(rows.at[pl.ds(0, n)],
                        acc_sh.at[pl.ds(s * RPTH + off, n)])


def _copy_out(rows, acc_sh, out_hbm, c, s):
    for off, n in ((0, CHUNK), (CHUNK, CHUNK), (2 * CHUNK, RPTH - 2 * CHUNK)):
        pltpu.sync_copy(acc_sh.at[pl.ds(s * RPTH + off, n)],
                        rows.at[pl.ds(0, n)])
        pltpu.sync_copy(rows.at[pl.ds(0, n)],
                        out_hbm.at[c, pl.ds(s * RPTH + off, n)])


def _remap_dst(didx, c):
    base = c * HALF

    def remap(t, carry):
        for j in range(CHUNK // 16):
            v = didx[t, pl.ds(j * 16, 16)] - base
            ok = (v >= 0) & (v < HALF)
            didx[t, pl.ds(j * 16, 16)] = jnp.where(ok, v, HALF)
        return carry

    lax.fori_loop(0, CPS, remap, 0)


@functools.partial(
    pl.kernel,
    out_type=jax.ShapeDtypeStruct((2, NROWH, F), jnp.float32),
    mesh=_mesh,
    scratch_types=[
        pltpu.VMEM((CPS, CHUNK), jnp.int32),
        pltpu.VMEM((CPS, CHUNK), jnp.int32),
        pltpu.VMEM((CHUNK, F), jnp.float32),
        pltpu.VMEM((CHUNK, F), jnp.float32),
        pltpu.VMEM_SHARED((NROWH, F), jnp.float32),
        pltpu.SemaphoreType.DMA,
        pltpu.SemaphoreType.DMA,
    ],
)
def _prop_kernel(g_hbm, src_hbm, dst_hbm, out_hbm, sidx, didx, rows0, rows1,
                 acc_sh, sem0, sem1):
    c = lax.axis_index("c")
    s = lax.axis_index("s")
    _zero_acc(rows0, acc_sh, s)
    plsc.subcore_barrier()

    pltpu.sync_copy(src_hbm.at[s], sidx)
    pltpu.sync_copy(dst_hbm.at[s], didx)
    _remap_dst(didx, c)

    pltpu.async_copy(g_hbm.at[sidx.at[0]], rows0, sem0)

    def body(i, carry):
        t = i * 2
        tn1 = t + 1
        tn2 = jnp.minimum(t + 2, CPS - 1)
        pltpu.async_copy(g_hbm.at[sidx.at[tn1]], rows1, sem1)
        pltpu.make_async_copy(g_hbm.at[sidx.at[t]], rows0, sem0).wait()
        pltpu.sync_copy(rows0, acc_sh.at[didx.at[t]], add=True)
        pltpu.async_copy(g_hbm.at[sidx.at[tn2]], rows0, sem0)
        pltpu.make_async_copy(g_hbm.at[sidx.at[tn1]], rows1, sem1).wait()
        pltpu.sync_copy(rows1, acc_sh.at[didx.at[tn1]], add=True)
        return carry

    lax.fori_loop(0, CPS // 2, body, 0)
    pltpu.make_async_copy(g_hbm.at[sidx.at[CPS - 1]], rows0, sem0).wait()
    plsc.subcore_barrier()
    _copy_out(rows0, acc_sh, out_hbm, c, s)


@functools.partial(
    pl.kernel,
    out_type=jax.ShapeDtypeStruct((2, NROWH, F), jnp.float32),
    mesh=_mesh,
    scratch_types=[
        pltpu.VMEM((CPS, CHUNK), jnp.int32),
        pltpu.VMEM((CHUNK, F), jnp.float32),
        pltpu.VMEM_SHARED((NROWH, F), jnp.float32),
    ],
)
def _deg_kernel(dst_hbm, out_hbm, didx, rows, acc_sh):
    c = lax.axis_index("c")
    s = lax.axis_index("s")
    _zero_acc(rows, acc_sh, s)
    plsc.subcore_barrier()

    pltpu.sync_copy(dst_hbm.at[s], didx)
    _remap_dst(didx, c)

    one_row = jnp.zeros((16,), jnp.float32) + 1.0

    def init_ones(i, carry):
        for j in range(F // 16):
            rows[i, pl.ds(j * 16, 16)] = one_row
        return carry

    lax.fori_loop(0, CHUNK, init_ones, 0)

    def body(t, carry):
        pltpu.sync_copy(rows, acc_sh.at[didx.at[t]], add=True)
        return carry

    lax.fori_loop(0, CPS, body, 0)
    plsc.subcore_barrier()
    _copy_out(rows, acc_sh, out_hbm, c, s)


def _tc1_body(x_ref, w_ref, degp_ref, g_ref, dinv_ref):
    d = jnp.concatenate(
        [degp_ref[0, :HALF, 0:1], degp_ref[1, :NN - HALF, 0:1]],
        axis=0) + 1.0
    dinv = lax.rsqrt(d)
    dinv_ref[...] = dinv
    h = jnp.dot(x_ref[...], w_ref[...], preferred_element_type=jnp.float32)
    g_ref[...] = dinv * h


_tc1 = pl.pallas_call(
    _tc1_body,
    out_shape=[
        jax.ShapeDtypeStruct((NN, F), jnp.float32),
        jax.ShapeDtypeStruct((NN, 1), jnp.float32),
    ],
)


def _agg(p_ref, g_ref):
    return jnp.concatenate(
        [p_ref[0, :HALF, :], p_ref[1, :NN - HALF, :]], axis=0) + g_ref[...]


def _tc_mid_body(p_ref, g_ref, dinv_ref, b_ref, w_ref, out_ref):
    xn = jnp.maximum(dinv_ref[...] * _agg(p_ref, g_ref) + b_ref[...], 0.0)
    h = jnp.dot(xn, w_ref[...], preferred_element_type=jnp.float32)
    out_ref[...] = dinv_ref[...] * h


_tc_mid = pl.pallas_call(
    _tc_mid_body,
    out_shape=jax.ShapeDtypeStruct((NN, F), jnp.float32),
)


def _tc_fin_body(p_ref, g_ref, dinv_ref, b_ref, batch_ref, wl_ref, bl_ref,
                 out_ref):
    x4 = dinv_ref[...] * _agg(p_ref, g_ref) + b_ref[...]
    gid = lax.broadcasted_iota(jnp.int32, (NG, NN), 0)
    m = (batch_ref[...] == gid).astype(jnp.float32)
    sums = jnp.dot(m, x4, preferred_element_type=jnp.float32)
    counts = jnp.sum(m, axis=1, keepdims=True)
    pooled = sums / jnp.maximum(counts, 1.0)
    out_ref[...] = (
        jnp.dot(pooled, wl_ref[...], preferred_element_type=jnp.float32)
        + bl_ref[...]
    )


_tc_fin = pl.pallas_call(
    _tc_fin_body,
    out_shape=jax.ShapeDtypeStruct((NG, F), jnp.float32),
)


def kernel(x, edge_index, batch, W1, b1, W2, b2, W3, b3, Wl, bl):
    src = edge_index[0].astype(jnp.int32)
    dst = edge_index[1].astype(jnp.int32)
    pad = EPAD - NE
    srcp = jnp.concatenate([src, jnp.zeros((pad,), jnp.int32)])
    srcp = srcp.reshape(NS, CPS, CHUNK)
    dstp = jnp.concatenate([dst, jnp.full((pad,), NN, jnp.int32)])
    dstp = dstp.reshape(NS, CPS, CHUNK)

    degp = _deg_kernel(dstp)
    g1, dinv = _tc1(x, W1, degp)
    p1 = _prop_kernel(g1, srcp, dstp)
    g2 = _tc_mid(p1, g1, dinv, b1.reshape(1, F), W2)
    p2 = _prop_kernel(g2, srcp, dstp)
    g3 = _tc_mid(p2, g2, dinv, b2.reshape(1, F), W3)
    p3 = _prop_kernel(g3, srcp, dstp)

    wlp = jnp.pad(Wl, ((0, 0), (0, F - NCLS)))
    blp = jnp.pad(bl, (0, F - NCLS)).reshape(1, F)
    out = _tc_fin(p3, g3, dinv, b3.reshape(1, F),
                  batch.astype(jnp.int32).reshape(1, NN), wlp, blp)
    return out[:, :NCLS]
